# baseline (speedup 1.0000x reference)
import functools

import jax
import jax.numpy as jnp
from jax import lax
from jax.experimental import pallas as pl
from jax.experimental.pallas import tpu as pltpu
from jax.experimental.pallas import tpu_sc as plsc

N_LEVELS = 256
THD_NEG = -128
NC = 2
NS = 16
NW = NC * NS
L = 16


def _sc_hard_body(qv_hbm, e_hbm, out_hbm, e_v, amax_v, shared_tbl, tbl_v,
                  idx_v, gout_v):
    cid = lax.axis_index("c")
    sid = lax.axis_index("s")
    wid = sid * NC + cid

    pltpu.sync_copy(e_hbm.at[pl.ds(sid * NS, NS)], e_v)
    lane = lax.broadcasted_iota(jnp.int32, (L,), 0)
    amax_acc = jnp.zeros((L,), jnp.float32)
    for r in range(NS):
        def chunk(c, carry):
            m, a = carry
            v = e_v[r, pl.ds(c * L, L)]
            upd = v > m
            a = jnp.where(upd, c * L + lane, a)
            m = jnp.maximum(m, v)
            return m, a

        m0 = jnp.full((L,), -jnp.inf, jnp.float32)
        a0 = jnp.zeros((L,), jnp.int32)
        m, a = lax.fori_loop(0, N_LEVELS // L, chunk, (m0, a0))
        big = m[0]
        amax_r = a[0]
        for i in range(1, L):
            mi = m[i]
            ai = a[i]
            gt = mi > big
            eq = mi == big
            amax_r = jnp.where(
                gt, ai, jnp.where(eq, jnp.minimum(amax_r, ai), amax_r)
            )
            big = jnp.maximum(big, mi)
        amax_acc = jnp.where(lane == r, amax_r.astype(jnp.float32), amax_acc)
    amax_v[...] = amax_acc
    pltpu.sync_copy(amax_v, shared_tbl.at[pl.ds(sid * NS, NS)])
    plsc.subcore_barrier()
    pltpu.sync_copy(shared_tbl, tbl_v)

    rows = N_LEVELS // NW
    pltpu.sync_copy(qv_hbm.at[pl.ds(wid * rows, rows)], idx_v)
    for r in range(rows):
        def gbody(c, carry):
            iv = idx_v[r, pl.ds(c * L, L)] - THD_NEG
            gout_v[r, pl.ds(c * L, L)] = plsc.load_gather(tbl_v, [iv])
            return carry

        lax.fori_loop(0, N_LEVELS // L, gbody, 0)
    pltpu.sync_copy(gout_v, out_hbm.at[pl.ds(wid * rows, rows)])


def _sc_hard(quantized_values, encoding_logits):
    n, d = quantized_values.shape
    mesh = plsc.VectorSubcoreMesh(core_axis_name="c", subcore_axis_name="s")
    rows = n // NW
    return pl.kernel(
        _sc_hard_body,
        out_type=jax.ShapeDtypeStruct((n, d), jnp.float32),
        mesh=mesh,
        scratch_types=[
            pltpu.VMEM((NS, N_LEVELS), jnp.float32),
            pltpu.VMEM((L,), jnp.float32),
            pltpu.VMEM_SHARED((N_LEVELS,), jnp.float32),
            pltpu.VMEM((N_LEVELS,), jnp.float32),
            pltpu.VMEM((rows, d), jnp.int32),
            pltpu.VMEM((rows, d), jnp.float32),
        ],
        compiler_params=pltpu.CompilerParams(needs_layout_passes=False),
    )(quantized_values, encoding_logits)


def _broadcast_body(h_ref, out_ref):
    out_ref[:] = jnp.broadcast_to(h_ref[:][None, :, :], out_ref.shape)


def kernel(quantized_values, encoding_logits):
    n, d = quantized_values.shape
    nl = encoding_logits.shape[0]
    hard = _sc_hard(quantized_values, encoding_logits)
    k_per_step = 16
    out = pl.pallas_call(
        _broadcast_body,
        grid=(nl // k_per_step,),
        in_specs=[pl.BlockSpec((n, d), lambda k: (0, 0))],
        out_specs=pl.BlockSpec((k_per_step, n, d), lambda k: (k, 0, 0)),
        out_shape=jax.ShapeDtypeStruct((nl, n, d), jnp.float32),
    )(hard)
    return out


# speedup vs baseline: 2.3170x; 2.3170x over previous
import jax
import jax.numpy as jnp
from jax.experimental import pallas as pl
from jax.experimental.pallas import tpu as pltpu

N_LEVELS = 256
THD_NEG = -128


def _fused_body(qv_ref, e_ref, out_ref, hard_ref):
    @pl.when(pl.program_id(0) == 0)
    def _():
        e = e_ref[:]
        m = jnp.max(e, axis=1, keepdims=True)
        col = jax.lax.broadcasted_iota(jnp.int32, e.shape, 1)
        amax = jnp.min(jnp.where(e == m, col, N_LEVELS), axis=1)
        amax_f = amax.astype(jnp.float32)
        n, d = qv_ref.shape
        idx = qv_ref[:] - THD_NEG
        half_w = 128
        parts = []
        for c in range(d // half_w):
            idxc = jax.lax.slice(idx, (0, c * half_w), (n, (c + 1) * half_w))
            acc = jnp.zeros((n, half_w), jnp.float32)
            for h in range(N_LEVELS // half_w):
                tbl = jnp.broadcast_to(
                    amax_f[None, h * half_w : (h + 1) * half_w], (n, half_w)
                )
                rel = jnp.clip(idxc - h * half_w, 0, half_w - 1)
                g = jnp.take_along_axis(tbl, rel, axis=1)
                acc = jnp.where(idxc // half_w == h, g, acc)
            parts.append(acc)
        hard_ref[:] = jnp.concatenate(parts, axis=1)

    out_ref[:] = jnp.broadcast_to(hard_ref[:][None, :, :], out_ref.shape)


def kernel(quantized_values, encoding_logits):
    n, d = quantized_values.shape
    nl = encoding_logits.shape[0]
    k_per_step = 16
    out = pl.pallas_call(
        _fused_body,
        grid=(nl // k_per_step,),
        in_specs=[
            pl.BlockSpec((n, d), lambda k: (0, 0)),
            pl.BlockSpec((nl, nl), lambda k: (0, 0)),
        ],
        out_specs=pl.BlockSpec((k_per_step, n, d), lambda k: (k, 0, 0)),
        out_shape=jax.ShapeDtypeStruct((nl, n, d), jnp.float32),
        scratch_shapes=[pltpu.VMEM((n, d), jnp.float32)],
    )(quantized_values, encoding_logits)
    return out
